# Initial kernel scaffold; baseline (speedup 1.0000x reference)
import functools

import jax
import jax.numpy as jnp
from jax import lax
from jax.experimental import pallas as pl
from jax.experimental.pallas import tpu as pltpu
from jax.experimental.pallas import tpu_sc as plsc

N = 10000
M = 320000
D_IN = 768
D = 128
L = 2

NC = 2
NS = 16
NW = NC * NS
PER_W = M // NW
CHUNK = 400
N_CHUNKS = PER_W // CHUNK
ROWS_PER_TILE = N // NS


def _lrelu(x):
    return jnp.where(x >= 0, x, 0.01 * x)


def _ln(x, g, b):
    mu = jnp.mean(x, axis=-1, keepdims=True)
    var = jnp.mean((x - mu) * (x - mu), axis=-1, keepdims=True)
    return (x - mu) * lax.rsqrt(var + 1e-5) * g + b


def _encoder(x, W, b, ln):
    BN = 1000

    def body(x_ref, w_ref, b_ref, ln_ref, o_ref):
        y = jnp.dot(x_ref[...], w_ref[...], preferred_element_type=jnp.float32)
        y = _lrelu(y + b_ref[...])
        o_ref[...] = _ln(y, ln_ref[0:1], ln_ref[1:2])

    return pl.pallas_call(
        body,
        grid=(N // BN,),
        in_specs=[
            pl.BlockSpec((BN, D_IN), lambda i: (i, 0)),
            pl.BlockSpec((D_IN, D), lambda i: (0, 0)),
            pl.BlockSpec((1, D), lambda i: (0, 0)),
            pl.BlockSpec((2, D), lambda i: (0, 0)),
        ],
        out_specs=pl.BlockSpec((BN, D), lambda i: (i, 0)),
        out_shape=jax.ShapeDtypeStruct((N, D), jnp.float32),
    )(x, W, b.reshape(1, D), ln)


def _edge(Hh, Ht, Eg, Ws, beu, bmf, bmb, lnr, rr=None, dir2=None, out_e=True):
    BE = 512
    first = rr is not None

    def body(*refs):
        if first:
            hh, ht, eg, rrr, d2, ws, b1, b2, b3, lnref = refs[:10]
            outs = refs[10:]
        else:
            hh, ht, eg, ws, b1, b2, b3, lnref = refs[:8]
            outs = refs[8:]
        Hh_ = hh[...]
        Ht_ = ht[...]
        E = eg[...]
        if first:
            w = rrr[...]
            E = E + d2[0:1, :] * (1.0 - w) + d2[1:2, :] * w
        dot = functools.partial(jnp.dot, preferred_element_type=jnp.float32)
        T = dot(Hh_, ws[0]) + dot(E, ws[1]) + dot(Ht_, ws[2]) + b1[...]
        En = _ln(_lrelu(T) + E, lnref[0:1], lnref[1:2])
        mf = dot(Hh_, ws[3]) + dot(En, ws[4]) + dot(Hh_ * En, ws[5]) + b2[...]
        mb = dot(Ht_, ws[6]) + dot(En, ws[7]) + dot(Ht_ * En, ws[8]) + b3[...]
        if out_e:
            outs[0][...] = En
            outs[1][...] = mf
            outs[2][...] = mb
        else:
            outs[0][...] = mf
            outs[1][...] = mb

    eb = pl.BlockSpec((BE, D), lambda i: (i, 0))

    def cb(shape):
        return pl.BlockSpec(shape, lambda i, _s=shape: tuple(0 for _ in _s))

    in_specs = [eb, eb, eb]
    args = [Hh, Ht, Eg]
    if first:
        in_specs += [pl.BlockSpec((BE, 1), lambda i: (i, 0)), cb((2, D))]
        args += [rr, dir2]
    in_specs += [cb((9, D, D)), cb((1, D)), cb((1, D)), cb((1, D)), cb((2, D))]
    args += [Ws, beu.reshape(1, D), bmf.reshape(1, D), bmb.reshape(1, D), lnr]
    n_out = 3 if out_e else 2
    out = pl.pallas_call(
        body,
        grid=(M // BE,),
        in_specs=in_specs,
        out_specs=[eb] * n_out,
        out_shape=[jax.ShapeDtypeStruct((M, D), jnp.float32)] * n_out,
    )(*args)
    return out


def _node(aggp, cntp3, H, lnr):
    BN = 1000

    def body(a_ref, c_ref, h_ref, ln_ref, o_ref):
        a = a_ref[0] + a_ref[1]
        cnt = c_ref[0] + c_ref[1]
        a = a / jnp.maximum(cnt, 1.0)
        x = _lrelu(a) + h_ref[...]
        o_ref[...] = _ln(x, ln_ref[0:1], ln_ref[1:2])

    return pl.pallas_call(
        body,
        grid=(N // BN,),
        in_specs=[
            pl.BlockSpec((2, BN, D), lambda i: (0, i, 0)),
            pl.BlockSpec((2, BN, 1), lambda i: (0, i, 0)),
            pl.BlockSpec((BN, D), lambda i: (i, 0)),
            pl.BlockSpec((2, D), lambda i: (0, 0)),
        ],
        out_specs=pl.BlockSpec((BN, D), lambda i: (i, 0)),
        out_shape=jax.ShapeDtypeStruct((N, D), jnp.float32),
    )(aggp, cntp3, H, lnr)


def _sc_gather(tables, idxs):
    n = len(tables)
    mesh = plsc.VectorSubcoreMesh(core_axis_name="c", subcore_axis_name="s")
    out_type = tuple(jax.ShapeDtypeStruct((M, D), jnp.float32) for _ in range(n))
    scratch = [
        pltpu.VMEM((CHUNK,), jnp.int32),
        pltpu.VMEM((CHUNK, D), jnp.float32),
        pltpu.SemaphoreType.DMA,
    ]

    def body(*refs):
        tbl = refs[:n]
        idx = refs[n:2 * n]
        out = refs[2 * n:3 * n]
        idx_v, rows_v, sem = refs[3 * n:]
        wid = lax.axis_index("s") * NC + lax.axis_index("c")
        base = wid * PER_W

        def step(i, carry):
            off = base + i * CHUNK
            for t in range(n):
                pltpu.sync_copy(idx[t].at[pl.ds(off, CHUNK)], idx_v)
                pltpu.async_copy(tbl[t].at[idx_v], rows_v, sem).wait()
                pltpu.sync_copy(rows_v, out[t].at[pl.ds(off, CHUNK)])
            return carry

        lax.fori_loop(0, N_CHUNKS, step, 0)

    f = pl.kernel(body, out_type=out_type, mesh=mesh, scratch_types=scratch)
    return f(*tables, *idxs)


def _sc_scatter(mf, mb, tails, heads, with_cnt):
    mesh = plsc.VectorSubcoreMesh(core_axis_name="c", subcore_axis_name="s")
    out_type = [jax.ShapeDtypeStruct((NC, N, D), jnp.float32)]
    scratch = [
        pltpu.VMEM_SHARED((N, D), jnp.float32),
        pltpu.VMEM((CHUNK, D), jnp.float32),
        pltpu.VMEM((CHUNK,), jnp.int32),
    ]
    zeros_blk = jnp.zeros((ROWS_PER_TILE, D), jnp.float32)
    args = [mf, mb, tails, heads, zeros_blk]
    if with_cnt:
        out_type.append(jax.ShapeDtypeStruct((NC, N), jnp.float32))
        scratch += [pltpu.VMEM_SHARED((N,), jnp.float32),
                    pltpu.VMEM((CHUNK,), jnp.float32)]
        args += [jnp.zeros((N,), jnp.float32), jnp.ones((CHUNK,), jnp.float32)]

    def body(*refs):
        if with_cnt:
            (mf_h, mb_h, t_h, h_h, zb_h, zn_h, on_h, agg_o, cnt_o,
             agg_s, buf_v, idx_v, cnt_s, ones_v) = refs
        else:
            mf_h, mb_h, t_h, h_h, zb_h, agg_o, agg_s, buf_v, idx_v = refs
        c = lax.axis_index("c")
        s = lax.axis_index("s")
        wid = s * NC + c
        r0 = s * ROWS_PER_TILE
        pltpu.sync_copy(zb_h, agg_s.at[pl.ds(r0, ROWS_PER_TILE)])
        if with_cnt:
            @pl.when(s == 0)
            def _():
                pltpu.sync_copy(zn_h, cnt_s)
            pltpu.sync_copy(on_h, ones_v)
        plsc.subcore_barrier()
        base = wid * PER_W

        def step(i, carry):
            off = base + i * CHUNK
            pltpu.sync_copy(t_h.at[pl.ds(off, CHUNK)], idx_v)
            pltpu.sync_copy(mf_h.at[pl.ds(off, CHUNK)], buf_v)
            pltpu.sync_copy(buf_v, agg_s.at[idx_v], add=True)
            if with_cnt:
                pltpu.sync_copy(ones_v, cnt_s.at[idx_v], add=True)
            pltpu.sync_copy(h_h.at[pl.ds(off, CHUNK)], idx_v)
            pltpu.sync_copy(mb_h.at[pl.ds(off, CHUNK)], buf_v)
            pltpu.sync_copy(buf_v, agg_s.at[idx_v], add=True)
            if with_cnt:
                pltpu.sync_copy(ones_v, cnt_s.at[idx_v], add=True)
            return carry

        lax.fori_loop(0, N_CHUNKS, step, 0)
        plsc.subcore_barrier()
        pltpu.sync_copy(agg_s.at[pl.ds(r0, ROWS_PER_TILE)],
                        agg_o.at[c, pl.ds(r0, ROWS_PER_TILE)])
        if with_cnt:
            @pl.when(s == 0)
            def _():
                pltpu.sync_copy(cnt_s, cnt_o.at[c])

    f = pl.kernel(body, out_type=tuple(out_type), mesh=mesh,
                  scratch_types=scratch)
    return f(*args)


def _layer_weights(W_eu, b_eu, W_mf, b_mf, W_mb, b_mb, ln_eu, l):
    Wh, We, Wt = W_eu[l, :D], W_eu[l, D:2 * D], W_eu[l, 2 * D:]
    Afh = W_mf[l, :D] + W_mf[l, 2 * D:3 * D]
    Afe = W_mf[l, D:2 * D] + W_mf[l, 2 * D:3 * D]
    Afp = W_mf[l, 3 * D:]
    Abt = W_mb[l, :D] + W_mb[l, 2 * D:3 * D]
    Abe = W_mb[l, D:2 * D] + W_mb[l, 2 * D:3 * D]
    Abp = W_mb[l, 3 * D:]
    Ws = jnp.stack([Wh, We, Wt, Afh, Afe, Afp, Abt, Abe, Abp])
    return Ws, b_eu[l], b_mf[l], b_mb[l], ln_eu[l]


def kernel(entity_feat, ht, r_tensor, r_relative, W_ent, b_ent, ln_ent,
           rel_emb, dir_emb, W_eu, b_eu, ln_eu, W_mf, b_mf, W_mb, b_mb,
           ln_mp):
    heads = ht[:, 0]
    tails = ht[:, 1]
    rr = r_relative.astype(jnp.float32).reshape(M, 1)

    H = _encoder(entity_feat, W_ent, b_ent, ln_ent)
    Hh, Ht, Erel = _sc_gather([H, H, rel_emb], [heads, tails, r_tensor])

    cnt3 = None
    E = None
    for l in range(L):
        Ws, beu, bmf, bmb, lnr = _layer_weights(
            W_eu, b_eu, W_mf, b_mf, W_mb, b_mb, ln_eu, l)
        if l == 0:
            E, mfm, mbm = _edge(Hh, Ht, Erel, Ws, beu, bmf, bmb, lnr,
                                rr=rr, dir2=dir_emb, out_e=True)
            aggp, cntp = _sc_scatter(mfm, mbm, tails, heads, True)
            cnt3 = cntp.reshape(NC, N, 1)
        else:
            mfm, mbm = _edge(Hh, Ht, E, Ws, beu, bmf, bmb, lnr, out_e=False)
            aggp = _sc_scatter(mfm, mbm, tails, heads, False)
        H = _node(aggp, cnt3, H, ln_mp[l])
        if l == 0:
            Hh, Ht = _sc_gather([H, H], [heads, tails])
    return H


# speedup vs baseline: 3.6234x; 3.6234x over previous
import functools

import jax
import jax.numpy as jnp
from jax import lax
from jax.experimental import pallas as pl
from jax.experimental.pallas import tpu as pltpu
from jax.experimental.pallas import tpu_sc as plsc

N = 10000
M = 320000
D_IN = 768
D = 128
L = 2

NC = 2
NS = 16
NW = NC * NS
PER_W = M // NW
CHUNK = 400
N_CHUNKS = PER_W // CHUNK
ROWS_PER_TILE = N // NS
SCHUNK = 200
S_CHUNKS = PER_W // SCHUNK


def _lrelu(x):
    return jnp.where(x >= 0, x, 0.01 * x)


def _ln(x, g, b):
    mu = jnp.mean(x, axis=-1, keepdims=True)
    var = jnp.mean((x - mu) * (x - mu), axis=-1, keepdims=True)
    return (x - mu) * lax.rsqrt(var + 1e-5) * g + b


def _encoder(x, W, b, ln):
    BN = 1000

    def body(x_ref, w_ref, b_ref, ln_ref, o_ref):
        y = jnp.dot(x_ref[...], w_ref[...], preferred_element_type=jnp.float32)
        y = _lrelu(y + b_ref[...])
        o_ref[...] = _ln(y, ln_ref[0:1], ln_ref[1:2])

    return pl.pallas_call(
        body,
        grid=(N // BN,),
        in_specs=[
            pl.BlockSpec((BN, D_IN), lambda i: (i, 0)),
            pl.BlockSpec((D_IN, D), lambda i: (0, 0)),
            pl.BlockSpec((1, D), lambda i: (0, 0)),
            pl.BlockSpec((2, D), lambda i: (0, 0)),
        ],
        out_specs=pl.BlockSpec((BN, D), lambda i: (i, 0)),
        out_shape=jax.ShapeDtypeStruct((N, D), jnp.float32),
    )(x, W, b.reshape(1, D), ln)


def _edge(Hh, Ht, Eg, Ws, beu, bmf, bmb, lnr, rr=None, dir2=None, out_e=True):
    BE = 512
    first = rr is not None

    def body(*refs):
        if first:
            hh, ht, eg, rrr, d2, ws, b1, b2, b3, lnref = refs[:10]
            outs = refs[10:]
        else:
            hh, ht, eg, ws, b1, b2, b3, lnref = refs[:8]
            outs = refs[8:]
        Hh_ = hh[...]
        Ht_ = ht[...]
        E = eg[...]
        if first:
            w = rrr[...]
            E = E + d2[0:1, :] * (1.0 - w) + d2[1:2, :] * w
        dot = functools.partial(jnp.dot, preferred_element_type=jnp.float32)
        T = dot(Hh_, ws[0]) + dot(E, ws[1]) + dot(Ht_, ws[2]) + b1[...]
        En = _ln(_lrelu(T) + E, lnref[0:1], lnref[1:2])
        mf = dot(Hh_, ws[3]) + dot(En, ws[4]) + dot(Hh_ * En, ws[5]) + b2[...]
        mb = dot(Ht_, ws[6]) + dot(En, ws[7]) + dot(Ht_ * En, ws[8]) + b3[...]
        if out_e:
            outs[0][...] = En
            outs[1][...] = mf
            outs[2][...] = mb
        else:
            outs[0][...] = mf
            outs[1][...] = mb

    eb = pl.BlockSpec((BE, D), lambda i: (i, 0))

    def cb(shape):
        return pl.BlockSpec(shape, lambda i, _s=shape: tuple(0 for _ in _s))

    in_specs = [eb, eb, eb]
    args = [Hh, Ht, Eg]
    if first:
        in_specs += [pl.BlockSpec((BE, 1), lambda i: (i, 0)), cb((2, D))]
        args += [rr, dir2]
    in_specs += [cb((9, D, D)), cb((1, D)), cb((1, D)), cb((1, D)), cb((2, D))]
    args += [Ws, beu.reshape(1, D), bmf.reshape(1, D), bmb.reshape(1, D), lnr]
    n_out = 3 if out_e else 2
    out = pl.pallas_call(
        body,
        grid=(M // BE,),
        in_specs=in_specs,
        out_specs=[eb] * n_out,
        out_shape=[jax.ShapeDtypeStruct((M, D), jnp.float32)] * n_out,
    )(*args)
    return out


def _node(aggp, cntp3, H, lnr):
    BN = 1000

    def body(a_ref, c_ref, h_ref, ln_ref, o_ref):
        a = a_ref[0] + a_ref[1]
        cnt = c_ref[0] + c_ref[1]
        a = a / jnp.maximum(cnt, 1.0)
        x = _lrelu(a) + h_ref[...]
        o_ref[...] = _ln(x, ln_ref[0:1], ln_ref[1:2])

    return pl.pallas_call(
        body,
        grid=(N // BN,),
        in_specs=[
            pl.BlockSpec((2, BN, D), lambda i: (0, i, 0)),
            pl.BlockSpec((2, BN, 1), lambda i: (0, i, 0)),
            pl.BlockSpec((BN, D), lambda i: (i, 0)),
            pl.BlockSpec((2, D), lambda i: (0, 0)),
        ],
        out_specs=pl.BlockSpec((BN, D), lambda i: (i, 0)),
        out_shape=jax.ShapeDtypeStruct((N, D), jnp.float32),
    )(aggp, cntp3, H, lnr)


def _sc_gather(tables, idxs):
    n = len(tables)
    mesh = plsc.VectorSubcoreMesh(core_axis_name="c", subcore_axis_name="s")
    out_type = tuple(jax.ShapeDtypeStruct((M, D), jnp.float32) for _ in range(n))
    scratch = [
        pltpu.VMEM((CHUNK,), jnp.int32),
        pltpu.VMEM((CHUNK, D), jnp.float32),
        pltpu.SemaphoreType.DMA,
    ]

    def body(*refs):
        tbl = refs[:n]
        idx = refs[n:2 * n]
        out = refs[2 * n:3 * n]
        idx_v, rows_v, sem = refs[3 * n:]
        wid = lax.axis_index("s") * NC + lax.axis_index("c")
        base = wid * PER_W

        def step(i, carry):
            off = base + i * CHUNK
            for t in range(n):
                pltpu.sync_copy(idx[t].at[pl.ds(off, CHUNK)], idx_v)
                pltpu.async_copy(tbl[t].at[idx_v], rows_v, sem).wait()
                pltpu.sync_copy(rows_v, out[t].at[pl.ds(off, CHUNK)])
            return carry

        lax.fori_loop(0, N_CHUNKS, step, 0)

    f = pl.kernel(body, out_type=out_type, mesh=mesh, scratch_types=scratch)
    return f(*tables, *idxs)


_WFULL = 632
_WLAST = N - (NS - 1) * _WFULL


def _sc_scatter(mf, mb, tails, heads, with_cnt):
    mesh = plsc.VectorSubcoreMesh(core_axis_name="c", subcore_axis_name="s")
    out_type = [jax.ShapeDtypeStruct((NC * N, D), jnp.float32)]
    scratch = [
        pltpu.VMEM_SHARED((N, D), jnp.float32),
        pltpu.VMEM((SCHUNK, D), jnp.float32),
        pltpu.VMEM((SCHUNK,), jnp.int32),
    ]
    zeros_blk = jnp.zeros((_WFULL, D), jnp.float32)
    args = [mf, mb, tails, heads, zeros_blk]
    if with_cnt:
        out_type.append(jax.ShapeDtypeStruct((NC * N,), jnp.float32))
        scratch += [pltpu.VMEM_SHARED((N,), jnp.float32),
                    pltpu.VMEM((SCHUNK,), jnp.float32),
                    pltpu.VMEM((N,), jnp.float32)]
        args += [jnp.zeros((N,), jnp.float32), jnp.ones((SCHUNK,), jnp.float32)]

    def body(*refs):
        if with_cnt:
            (mf_h, mb_h, t_h, h_h, zb_h, zn_h, on_h, agg_o, cnt_o,
             agg_s, buf_v, idx_v, cnt_s, ones_v, cnt_v) = refs
        else:
            mf_h, mb_h, t_h, h_h, zb_h, agg_o, agg_s, buf_v, idx_v = refs
        c = lax.axis_index("c")
        s = lax.axis_index("s")
        wid = s * NC + c
        r0 = pl.multiple_of(s * _WFULL, 8)

        @pl.when(s < NS - 1)
        def _():
            pltpu.sync_copy(zb_h, agg_s.at[pl.ds(r0, _WFULL)])

        @pl.when(s == NS - 1)
        def _():
            pltpu.sync_copy(zb_h.at[pl.ds(0, _WLAST)],
                            agg_s.at[pl.ds(r0, _WLAST)])

        if with_cnt:
            @pl.when(s == 0)
            def _():
                pltpu.sync_copy(zn_h, cnt_v)
                pltpu.sync_copy(cnt_v, cnt_s)
            pltpu.sync_copy(on_h, ones_v)
        plsc.subcore_barrier()
        base = wid * PER_W

        def step(i, carry):
            off = pl.multiple_of(base + i * SCHUNK, 8)
            pltpu.sync_copy(t_h.at[pl.ds(off, SCHUNK)], idx_v)
            pltpu.sync_copy(mf_h.at[pl.ds(off, SCHUNK)], buf_v)
            pltpu.sync_copy(buf_v, agg_s.at[idx_v], add=True)
            if with_cnt:
                pltpu.sync_copy(ones_v, cnt_s.at[idx_v], add=True)
            pltpu.sync_copy(h_h.at[pl.ds(off, SCHUNK)], idx_v)
            pltpu.sync_copy(mb_h.at[pl.ds(off, SCHUNK)], buf_v)
            pltpu.sync_copy(buf_v, agg_s.at[idx_v], add=True)
            if with_cnt:
                pltpu.sync_copy(ones_v, cnt_s.at[idx_v], add=True)
            return carry

        lax.fori_loop(0, S_CHUNKS, step, 0)
        plsc.subcore_barrier()
        o0 = pl.multiple_of(c * N + r0, 8)

        @pl.when(s < NS - 1)
        def _():
            pltpu.sync_copy(agg_s.at[pl.ds(r0, _WFULL)],
                            agg_o.at[pl.ds(o0, _WFULL)])

        @pl.when(s == NS - 1)
        def _():
            pltpu.sync_copy(agg_s.at[pl.ds(r0, _WLAST)],
                            agg_o.at[pl.ds(o0, _WLAST)])

        if with_cnt:
            @pl.when(s == 0)
            def _():
                pltpu.sync_copy(cnt_s, cnt_v)
                pltpu.sync_copy(cnt_v,
                                cnt_o.at[pl.ds(pl.multiple_of(c * N, 8), N)])

    f = pl.kernel(body, out_type=tuple(out_type), mesh=mesh,
                  scratch_types=scratch)
    res = f(*args)
    return res if with_cnt else res[0]


def _layer_weights(W_eu, b_eu, W_mf, b_mf, W_mb, b_mb, ln_eu, l):
    Wh, We, Wt = W_eu[l, :D], W_eu[l, D:2 * D], W_eu[l, 2 * D:]
    Afh = W_mf[l, :D] + W_mf[l, 2 * D:3 * D]
    Afe = W_mf[l, D:2 * D] + W_mf[l, 2 * D:3 * D]
    Afp = W_mf[l, 3 * D:]
    Abt = W_mb[l, :D] + W_mb[l, 2 * D:3 * D]
    Abe = W_mb[l, D:2 * D] + W_mb[l, 2 * D:3 * D]
    Abp = W_mb[l, 3 * D:]
    Ws = jnp.stack([Wh, We, Wt, Afh, Afe, Afp, Abt, Abe, Abp])
    return Ws, b_eu[l], b_mf[l], b_mb[l], ln_eu[l]


def kernel(entity_feat, ht, r_tensor, r_relative, W_ent, b_ent, ln_ent,
           rel_emb, dir_emb, W_eu, b_eu, ln_eu, W_mf, b_mf, W_mb, b_mb,
           ln_mp):
    heads = ht[:, 0]
    tails = ht[:, 1]
    rr = r_relative.astype(jnp.float32).reshape(M, 1)

    H = _encoder(entity_feat, W_ent, b_ent, ln_ent)
    Hh, Ht, Erel = _sc_gather([H, H, rel_emb], [heads, tails, r_tensor])

    cnt3 = None
    E = None
    for l in range(L):
        Ws, beu, bmf, bmb, lnr = _layer_weights(
            W_eu, b_eu, W_mf, b_mf, W_mb, b_mb, ln_eu, l)
        if l == 0:
            E, mfm, mbm = _edge(Hh, Ht, Erel, Ws, beu, bmf, bmb, lnr,
                                rr=rr, dir2=dir_emb, out_e=True)
            aggp, cntp = _sc_scatter(mfm, mbm, tails, heads, True)
            cnt3 = cntp.reshape(NC, N, 1)
        else:
            mfm, mbm = _edge(Hh, Ht, E, Ws, beu, bmf, bmb, lnr, out_e=False)
            aggp = _sc_scatter(mfm, mbm, tails, heads, False)
        H = _node(aggp.reshape(NC, N, D), cnt3, H, ln_mp[l])
        if l == 0:
            Hh, Ht = _sc_gather([H, H], [heads, tails])
    return H
